# Initial kernel scaffold; baseline (speedup 1.0000x reference)
import jax
import jax.numpy as jnp
from jax.experimental import pallas as pl


def kernel(x, feats, ln_gamma, ln_beta, W_feat, b_feat, W_r1, b_r1, W_r2, b_r2, W_e1h, W_e1f, b_e1, W_e2, b_e2):
    raise NotImplementedError("write your pallas kernel here")


# speedup vs baseline: 2.5794x; 2.5794x over previous
import functools

import jax
import jax.numpy as jnp
import numpy as np
from jax.experimental import pallas as pl
from jax.experimental.pallas import tpu as pltpu

_T = 8192
_D = 1024
_NF = 16
_DFE = 64
_DRH = 128
_DEH = 256
_E = 8
_BT = 256


def _moe_body(x_ref, feats_ref, g_ref, b_ref, wf_ref, bf_ref, wr1_ref, br1_ref,
              wr2_ref, br2_ref, we1h_ref, we1f_ref, be1_ref, we2_ref, be2_ref,
              y_ref):
    f32 = jnp.float32
    x = x_ref[...]
    mu = jnp.mean(x, axis=-1, keepdims=True)
    xc = x - mu
    var = jnp.mean(xc * xc, axis=-1, keepdims=True)
    h = xc * jax.lax.rsqrt(var + 1e-5) * g_ref[...] + b_ref[...]

    feats = feats_ref[...]
    feat_emb = jnp.dot(feats, wf_ref[...], preferred_element_type=f32) + bf_ref[...]
    r_h = jnp.dot(h, wr1_ref[:_D, :], preferred_element_type=f32)
    r_h += jnp.dot(feat_emb, wr1_ref[_D:, :], preferred_element_type=f32)
    r_h = jnp.maximum(r_h + br1_ref[...], 0.0)
    logits = jnp.dot(r_h, wr2_ref[...], preferred_element_type=f32) + br2_ref[...]

    eidx = jax.lax.broadcasted_iota(jnp.int32, logits.shape, 1)
    m1 = jnp.max(logits, axis=-1, keepdims=True)
    i1 = jnp.min(jnp.where(logits >= m1, eidx, _E), axis=-1, keepdims=True)
    masked = jnp.where(eidx == i1, -jnp.inf, logits)
    m2 = jnp.max(masked, axis=-1, keepdims=True)
    i2 = jnp.min(jnp.where(masked >= m2, eidx, _E), axis=-1, keepdims=True)
    eb = jnp.exp(m2 - m1)
    denom = 1.0 + eb
    w1 = 1.0 / denom
    w2 = eb / denom
    weights = (jnp.where(eidx == i1, w1, 0.0)
               + jnp.where(eidx == i2, w2, 0.0))

    hb = h.astype(jnp.bfloat16)
    acc = jnp.dot(weights, be2_ref[...], preferred_element_type=f32)
    for e in range(_E):
        ef = feats[:, 4 * (e // 2):4 * (e // 2) + 4]
        h1 = jnp.dot(hb, we1h_ref[e], preferred_element_type=f32)
        h1 += jnp.dot(ef, we1f_ref[e], preferred_element_type=f32)
        h1 = jnp.maximum(h1 + be1_ref[e][None, :], 0.0).astype(jnp.bfloat16)
        out_e = jnp.dot(h1, we2_ref[e], preferred_element_type=f32)
        acc += weights[:, e][:, None] * out_e
    y_ref[...] = x + acc


@jax.jit
def kernel(x, feats, ln_gamma, ln_beta, W_feat, b_feat, W_r1, b_r1, W_r2, b_r2,
           W_e1h, W_e1f, b_e1, W_e2, b_e2):
    tile = lambda i: (i, 0)
    whole = lambda i: (0, 0)
    whole3 = lambda i: (0, 0, 0)
    grid = _T // _BT
    out = pl.pallas_call(
        _moe_body,
        grid=(grid,),
        in_specs=[
            pl.BlockSpec((_BT, _D), tile),
            pl.BlockSpec((_BT, _NF), tile),
            pl.BlockSpec((1, _D), whole),
            pl.BlockSpec((1, _D), whole),
            pl.BlockSpec((_NF, _DFE), whole),
            pl.BlockSpec((1, _DFE), whole),
            pl.BlockSpec((_D + _DFE, _DRH), whole),
            pl.BlockSpec((1, _DRH), whole),
            pl.BlockSpec((_DRH, _E), whole),
            pl.BlockSpec((1, _E), whole),
            pl.BlockSpec((_E, _D, _DEH), whole3),
            pl.BlockSpec((_E, 4, _DEH), whole3),
            pl.BlockSpec((_E, _DEH), whole),
            pl.BlockSpec((_E, _DEH, _D), whole3),
            pl.BlockSpec((_E, _D), whole),
        ],
        out_specs=pl.BlockSpec((_BT, _D), tile),
        out_shape=jax.ShapeDtypeStruct((_T, _D), jnp.float32),
        compiler_params=pltpu.CompilerParams(
            dimension_semantics=("arbitrary",),
        ),
    )(
        x, feats,
        ln_gamma.reshape(1, _D), ln_beta.reshape(1, _D),
        W_feat, b_feat.reshape(1, _DFE),
        W_r1, b_r1.reshape(1, _DRH),
        W_r2, b_r2.reshape(1, _E),
        W_e1h.astype(jnp.bfloat16), W_e1f, b_e1,
        W_e2.astype(jnp.bfloat16), b_e2,
    )
    return out


# speedup vs baseline: 3.5482x; 1.3756x over previous
import jax
import jax.numpy as jnp
import numpy as np
from jax.experimental import pallas as pl
from jax.experimental.pallas import tpu as pltpu

_T = 8192
_D = 1024
_NF = 16
_DFE = 64
_DRH = 128
_DEH = 256
_E = 8
_EH = _E * _DEH
_BT = 256

_R_EXPAND = np.kron(np.eye(_E, dtype=np.float32), np.ones((1, _DEH), np.float32))


def _moe_body(x_ref, feats_ref, g_ref, b_ref, wf_ref, bf_ref, wr1_ref, br1_ref,
              wr2_ref, br2_ref, w1h_ref, w1f_ref, b1_ref, w2_ref, be2_ref,
              rexp_ref, y_ref):
    f32 = jnp.float32
    x = x_ref[...]
    mu = jnp.mean(x, axis=-1, keepdims=True)
    xc = x - mu
    var = jnp.mean(xc * xc, axis=-1, keepdims=True)
    h = xc * jax.lax.rsqrt(var + 1e-5) * g_ref[...] + b_ref[...]

    feats = feats_ref[...]
    feat_emb = jnp.dot(feats, wf_ref[...], preferred_element_type=f32) + bf_ref[...]
    r_h = jnp.dot(h, wr1_ref[:_D, :], preferred_element_type=f32)
    r_h += jnp.dot(feat_emb, wr1_ref[_D:, :], preferred_element_type=f32)
    r_h = jnp.maximum(r_h + br1_ref[...], 0.0)
    logits = jnp.dot(r_h, wr2_ref[...], preferred_element_type=f32) + br2_ref[...]

    eidx = jax.lax.broadcasted_iota(jnp.int32, logits.shape, 1)
    m1 = jnp.max(logits, axis=-1, keepdims=True)
    i1 = jnp.min(jnp.where(logits >= m1, eidx, _E), axis=-1, keepdims=True)
    masked = jnp.where(eidx == i1, -jnp.inf, logits)
    m2 = jnp.max(masked, axis=-1, keepdims=True)
    i2 = jnp.min(jnp.where(masked >= m2, eidx, _E), axis=-1, keepdims=True)
    eb = jnp.exp(m2 - m1)
    denom = 1.0 + eb
    w1 = 1.0 / denom
    w2 = eb / denom
    weights = (jnp.where(eidx == i1, w1, 0.0)
               + jnp.where(eidx == i2, w2, 0.0))

    hb = h.astype(jnp.bfloat16)
    h1 = jnp.dot(hb, w1h_ref[...], preferred_element_type=f32)
    h1 += jnp.dot(feats, w1f_ref[...], preferred_element_type=f32)
    h1 = jnp.maximum(h1 + b1_ref[...], 0.0)
    w_rep = jnp.dot(weights, rexp_ref[...], preferred_element_type=f32)
    h1s = (h1 * w_rep).astype(jnp.bfloat16)
    acc = jnp.dot(h1s, w2_ref[...], preferred_element_type=f32)
    acc += jnp.dot(weights, be2_ref[...], preferred_element_type=f32)
    y_ref[...] = x + acc


@jax.jit
def kernel(x, feats, ln_gamma, ln_beta, W_feat, b_feat, W_r1, b_r1, W_r2, b_r2,
           W_e1h, W_e1f, b_e1, W_e2, b_e2):
    tile = lambda i: (i, 0)
    whole = lambda i: (0, 0)
    grid = _T // _BT

    w1h_all = jnp.transpose(W_e1h, (1, 0, 2)).reshape(_D, _EH).astype(jnp.bfloat16)
    w1f_all = jnp.zeros((_NF, _E, _DEH), jnp.float32)
    for e in range(_E):
        w1f_all = w1f_all.at[4 * (e // 2):4 * (e // 2) + 4, e, :].set(W_e1f[e])
    w1f_all = w1f_all.reshape(_NF, _EH)
    b1_all = b_e1.reshape(1, _EH)
    w2_all = W_e2.reshape(_EH, _D).astype(jnp.bfloat16)

    out = pl.pallas_call(
        _moe_body,
        grid=(grid,),
        in_specs=[
            pl.BlockSpec((_BT, _D), tile),
            pl.BlockSpec((_BT, _NF), tile),
            pl.BlockSpec((1, _D), whole),
            pl.BlockSpec((1, _D), whole),
            pl.BlockSpec((_NF, _DFE), whole),
            pl.BlockSpec((1, _DFE), whole),
            pl.BlockSpec((_D + _DFE, _DRH), whole),
            pl.BlockSpec((1, _DRH), whole),
            pl.BlockSpec((_DRH, _E), whole),
            pl.BlockSpec((1, _E), whole),
            pl.BlockSpec((_D, _EH), whole),
            pl.BlockSpec((_NF, _EH), whole),
            pl.BlockSpec((1, _EH), whole),
            pl.BlockSpec((_EH, _D), whole),
            pl.BlockSpec((_E, _D), whole),
            pl.BlockSpec((_E, _EH), whole),
        ],
        out_specs=pl.BlockSpec((_BT, _D), tile),
        out_shape=jax.ShapeDtypeStruct((_T, _D), jnp.float32),
        compiler_params=pltpu.CompilerParams(
            dimension_semantics=("arbitrary",),
        ),
    )(
        x, feats,
        ln_gamma.reshape(1, _D), ln_beta.reshape(1, _D),
        W_feat, b_feat.reshape(1, _DFE),
        W_r1, b_r1.reshape(1, _DRH),
        W_r2, b_r2.reshape(1, _E),
        w1h_all, w1f_all, b1_all, w2_all, b_e2,
        jnp.asarray(_R_EXPAND),
    )
    return out


# speedup vs baseline: 3.7508x; 1.0571x over previous
import jax
import jax.numpy as jnp
import numpy as np
from jax.experimental import pallas as pl
from jax.experimental.pallas import tpu as pltpu

_T = 8192
_D = 1024
_NF = 16
_DFE = 64
_DRH = 128
_DEH = 256
_E = 8
_EH = _E * _DEH
_BT = 512

_R_EXPAND = np.kron(np.eye(_E, dtype=np.float32), np.ones((1, _DEH), np.float32))


def _moe_body(x_ref, feats_ref, g_ref, b_ref, wf_ref, bf_ref, wr1_ref, br1_ref,
              wr2_ref, br2_ref, w1h_ref, w1f_ref, b1_ref, w2_ref, be2_ref,
              rexp_ref, y_ref):
    f32 = jnp.float32
    x = x_ref[...]
    mu = jnp.mean(x, axis=-1, keepdims=True)
    xc = x - mu
    var = jnp.mean(xc * xc, axis=-1, keepdims=True)
    h = xc * jax.lax.rsqrt(var + 1e-5) * g_ref[...] + b_ref[...]

    feats = feats_ref[...]
    feat_emb = jnp.dot(feats, wf_ref[...], preferred_element_type=f32) + bf_ref[...]
    r_h = jnp.dot(h, wr1_ref[:_D, :], preferred_element_type=f32)
    r_h += jnp.dot(feat_emb, wr1_ref[_D:, :], preferred_element_type=f32)
    r_h = jnp.maximum(r_h + br1_ref[...], 0.0)
    logits = jnp.dot(r_h, wr2_ref[...], preferred_element_type=f32) + br2_ref[...]

    eidx = jax.lax.broadcasted_iota(jnp.int32, logits.shape, 1)
    m1 = jnp.max(logits, axis=-1, keepdims=True)
    i1 = jnp.min(jnp.where(logits >= m1, eidx, _E), axis=-1, keepdims=True)
    masked = jnp.where(eidx == i1, -jnp.inf, logits)
    m2 = jnp.max(masked, axis=-1, keepdims=True)
    i2 = jnp.min(jnp.where(masked >= m2, eidx, _E), axis=-1, keepdims=True)
    eb = jnp.exp(m2 - m1)
    denom = 1.0 + eb
    w1 = 1.0 / denom
    w2 = eb / denom
    weights = (jnp.where(eidx == i1, w1, 0.0)
               + jnp.where(eidx == i2, w2, 0.0))

    hb = h.astype(jnp.bfloat16)
    h1 = jnp.dot(hb, w1h_ref[...], preferred_element_type=f32)
    h1 += jnp.dot(feats, w1f_ref[...], preferred_element_type=f32)
    h1 = jnp.maximum(h1 + b1_ref[...], 0.0)
    w_rep = jnp.dot(weights, rexp_ref[...], preferred_element_type=f32)
    h1s = (h1 * w_rep).astype(jnp.bfloat16)
    acc = jnp.dot(h1s, w2_ref[...], preferred_element_type=f32)
    acc += jnp.dot(weights, be2_ref[...], preferred_element_type=f32)
    y_ref[...] = x + acc


@jax.jit
def kernel(x, feats, ln_gamma, ln_beta, W_feat, b_feat, W_r1, b_r1, W_r2, b_r2,
           W_e1h, W_e1f, b_e1, W_e2, b_e2):
    tile = lambda i: (i, 0)
    whole = lambda i: (0, 0)
    grid = _T // _BT

    w1h_all = jnp.transpose(W_e1h, (1, 0, 2)).reshape(_D, _EH).astype(jnp.bfloat16)
    w1f_all = jnp.zeros((_NF, _E, _DEH), jnp.float32)
    for e in range(_E):
        w1f_all = w1f_all.at[4 * (e // 2):4 * (e // 2) + 4, e, :].set(W_e1f[e])
    w1f_all = w1f_all.reshape(_NF, _EH)
    b1_all = b_e1.reshape(1, _EH)
    w2_all = W_e2.reshape(_EH, _D).astype(jnp.bfloat16)

    out = pl.pallas_call(
        _moe_body,
        grid=(grid,),
        in_specs=[
            pl.BlockSpec((_BT, _D), tile),
            pl.BlockSpec((_BT, _NF), tile),
            pl.BlockSpec((1, _D), whole),
            pl.BlockSpec((1, _D), whole),
            pl.BlockSpec((_NF, _DFE), whole),
            pl.BlockSpec((1, _DFE), whole),
            pl.BlockSpec((_D + _DFE, _DRH), whole),
            pl.BlockSpec((1, _DRH), whole),
            pl.BlockSpec((_DRH, _E), whole),
            pl.BlockSpec((1, _E), whole),
            pl.BlockSpec((_D, _EH), whole),
            pl.BlockSpec((_NF, _EH), whole),
            pl.BlockSpec((1, _EH), whole),
            pl.BlockSpec((_EH, _D), whole),
            pl.BlockSpec((_E, _D), whole),
            pl.BlockSpec((_E, _EH), whole),
        ],
        out_specs=pl.BlockSpec((_BT, _D), tile),
        out_shape=jax.ShapeDtypeStruct((_T, _D), jnp.float32),
        compiler_params=pltpu.CompilerParams(
            dimension_semantics=("arbitrary",),
        ),
    )(
        x, feats,
        ln_gamma.reshape(1, _D), ln_beta.reshape(1, _D),
        W_feat, b_feat.reshape(1, _DFE),
        W_r1, b_r1.reshape(1, _DRH),
        W_r2, b_r2.reshape(1, _E),
        w1h_all, w1f_all, b1_all, w2_all, b_e2,
        jnp.asarray(_R_EXPAND),
    )
    return out
